# Initial kernel scaffold; baseline (speedup 1.0000x reference)
import jax
import jax.numpy as jnp
from jax.experimental import pallas as pl


def kernel(src, source_lengths, params):
    raise NotImplementedError("write your pallas kernel here")


# speedup vs baseline: 3.7513x; 3.7513x over previous
import functools
import math

import jax
import jax.numpy as jnp
import numpy as np
from jax import lax
from jax.experimental import pallas as pl
from jax.experimental.pallas import tpu as pltpu
from jax.experimental.pallas import tpu_sc as plsc

VOCAB = 50000; D = 768; H = 12; DH = 64; S = 2048; B = 2
NH = 4; BK = 64; NB = 32; NCLS = 50
N = B * H
NR = N * NH
ROWS = N * S
SROWS = NR * S
RW = 256
OW = 128
TS = 256
NC_CHUNKS = S // BK
NSC = 32
HI = lax.Precision.HIGHEST


def _pe_table():
    pos = np.arange(S)[:, None].astype(np.float32)
    div = np.exp(np.arange(0, D, 2).astype(np.float32) * (-np.log(10000.0) / D))
    pe = np.zeros((S, D), dtype=np.float32)
    pe[:, 0::2] = np.sin(pos * div)
    pe[:, 1::2] = np.cos(pos * div)
    return jnp.asarray(pe)

PE = _pe_table()


def _sc_mesh():
    return plsc.VectorSubcoreMesh(core_axis_name="c", subcore_axis_name="s")


def _emb_gather(emb, src_flat):
    per_w = (B * S) // NSC

    @functools.partial(
        pl.kernel, mesh=_sc_mesh(),
        out_type=jax.ShapeDtypeStruct((B * S, D), jnp.float32),
        scratch_types=[pltpu.VMEM((per_w,), jnp.int32),
                       pltpu.VMEM((per_w, D), jnp.float32),
                       pltpu.SemaphoreType.DMA])
    def k(emb_hbm, src_hbm, out_hbm, idx_v, rows_v, sem):
        wid = lax.axis_index("s") * 2 + lax.axis_index("c")
        base = wid * per_w
        pltpu.sync_copy(src_hbm.at[pl.ds(base, per_w)], idx_v)
        pltpu.async_copy(emb_hbm.at[idx_v], rows_v, sem).wait()
        pltpu.sync_copy(rows_v, out_hbm.at[pl.ds(base, per_w)])

    return k(emb, src_flat)


def _sc_scatter(qkvp_flat, undo_flat):
    per_w = SROWS // NSC
    ch = 256
    nch = per_w // ch

    @functools.partial(
        pl.kernel, mesh=_sc_mesh(),
        out_type=jax.ShapeDtypeStruct((SROWS, RW), jnp.float32),
        scratch_types=[pltpu.VMEM((ch,), jnp.int32),
                       pltpu.VMEM((ch, RW), jnp.float32),
                       pltpu.SemaphoreType.DMA])
    def k(rows_hbm, undo_hbm, out_hbm, idx_v, rows_v, sem):
        wid = lax.axis_index("s") * 2 + lax.axis_index("c")

        def body(c, carry):
            t0 = wid * per_w + c * ch
            nj = t0 // S
            i0 = t0 - nj * S
            src0 = (nj // NH) * S + i0
            pltpu.sync_copy(rows_hbm.at[pl.ds(src0, ch)], rows_v)
            pltpu.sync_copy(undo_hbm.at[pl.ds(t0, ch)], idx_v)
            pltpu.async_copy(rows_v, out_hbm.at[idx_v], sem).wait()
            return carry

        lax.fori_loop(0, nch, body, 0)

    return k(qkvp_flat, undo_flat)


def _sc_gather(so_flat, undo_flat):
    per_w = SROWS // NSC
    ch = 512
    nch = per_w // ch

    @functools.partial(
        pl.kernel, mesh=_sc_mesh(),
        out_type=jax.ShapeDtypeStruct((SROWS, OW), jnp.float32),
        scratch_types=[pltpu.VMEM((ch,), jnp.int32),
                       pltpu.VMEM((ch, OW), jnp.float32),
                       pltpu.SemaphoreType.DMA])
    def k(so_hbm, undo_hbm, out_hbm, idx_v, rows_v, sem):
        wid = lax.axis_index("s") * 2 + lax.axis_index("c")

        def body(c, carry):
            t0 = wid * per_w + c * ch
            pltpu.sync_copy(undo_hbm.at[pl.ds(t0, ch)], idx_v)
            pltpu.async_copy(so_hbm.at[idx_v], rows_v, sem).wait()
            pltpu.sync_copy(rows_v, out_hbm.at[pl.ds(t0, ch)])
            return carry

        lax.fori_loop(0, nch, body, 0)

    return k(so_flat, undo_flat)


def _layer_norm(x, g, b):
    m = jnp.mean(x, axis=-1, keepdims=True)
    v = jnp.mean((x - m) ** 2, axis=-1, keepdims=True)
    return (x - m) / jnp.sqrt(v + 1e-5) * g + b


def _qkv_rows(x, srcf, p):
    def body(x_ref, srcf_ref, g_ref, b_ref, wqk_ref, wv_ref, o_ref):
        sblk = pl.program_id(1)
        xb = x_ref[0]
        xn = _layer_norm(xb, g_ref[...], b_ref[...])
        qk = jnp.dot(xn, wqk_ref[0])
        ss = jnp.sum(qk * qk, axis=-1, keepdims=True)
        kn = qk / (jnp.sqrt(ss) + 1e-9)
        v = jnp.dot(xn, wv_ref[0])
        padf = (srcf_ref[0] == 0.0).astype(jnp.float32)
        posf = (sblk * TS).astype(jnp.float32) + \
            lax.broadcasted_iota(jnp.int32, (TS, 1), 0).astype(jnp.float32)
        lane = lax.broadcasted_iota(jnp.int32, (TS, 64), 1)
        extra = jnp.where(lane == 0, padf, 0.0) + jnp.where(lane == 1, posf, 0.0)
        o_ref[0, 0] = jnp.concatenate([qk, kn, v, extra], axis=1)

    return pl.pallas_call(
        body,
        grid=(B, S // TS, H),
        in_specs=[
            pl.BlockSpec((1, TS, D), lambda b, s, h: (b, s, 0)),
            pl.BlockSpec((1, TS, 1), lambda b, s, h: (b, s, 0)),
            pl.BlockSpec((1, D), lambda b, s, h: (0, 0)),
            pl.BlockSpec((1, D), lambda b, s, h: (0, 0)),
            pl.BlockSpec((1, D, DH), lambda b, s, h: (h, 0, 0)),
            pl.BlockSpec((1, D, DH), lambda b, s, h: (h, 0, 0)),
        ],
        out_specs=pl.BlockSpec((1, 1, TS, RW), lambda b, s, h: (b, h, s, 0)),
        out_shape=jax.ShapeDtypeStruct((B, H, S, RW), jnp.float32),
    )(x, srcf, p['ln1_g'].reshape(1, D), p['ln1_b'].reshape(1, D),
      p['Wqk'].reshape(D, H, DH).transpose(1, 0, 2),
      p['Wv'].reshape(D, H, DH).transpose(1, 0, 2))


def _ranks(qkvp, rot2):
    SUB = 128
    NSUB = S // SUB

    def body(qk_ref, rot_ref, o_ref):
        b = pl.program_id(0)
        h = pl.program_id(1)
        qk = qk_ref[0, 0, :, 0:DH]
        rall = jnp.dot(qk, rot_ref[...])
        tri = (lax.broadcasted_iota(jnp.int32, (SUB, SUB), 0) >=
               lax.broadcasted_iota(jnp.int32, (SUB, SUB), 1)).astype(jnp.float32)
        lane32 = lax.broadcasted_iota(jnp.int32, (SUB, NB), 1)
        for j in range(NH):
            r = rall[:, j * 16:(j + 1) * 16]
            rc = jnp.concatenate([r, -r], axis=1)
            mx = jnp.max(rc, axis=1, keepdims=True)
            l2 = lax.broadcasted_iota(jnp.int32, (S, NB), 1)
            bucket = jnp.min(jnp.where(rc == mx, l2, NB),
                             axis=1, keepdims=True)
            bucketf = bucket.astype(jnp.float32)
            run = jnp.zeros((1, NB), jnp.float32)
            withins = []
            for kb in range(NSUB):
                bkb = bucket[kb * SUB:(kb + 1) * SUB]
                ob = (lane32 == bkb).astype(jnp.float32)
                inc = jnp.dot(tri, ob, precision=HI) + run
                withins.append(jnp.sum(inc * ob, axis=1, keepdims=True) - 1.0)
                run = run + jnp.sum(ob, axis=0, keepdims=True)
            base = (((b * H + h) * NH + j) * S).astype(jnp.float32)
            for kb in range(NSUB):
                bf = bucketf[kb * SUB:(kb + 1) * SUB]
                mlt = (lane32.astype(jnp.float32) < bf).astype(jnp.float32)
                off = jnp.sum(mlt * run, axis=1, keepdims=True)
                rank = off + withins[kb] + base
                o_ref[0, 0, j, kb * SUB:(kb + 1) * SUB] = rank.astype(jnp.int32)

    return pl.pallas_call(
        body,
        grid=(B, H),
        in_specs=[
            pl.BlockSpec((1, 1, S, 2 * DH), lambda b, h: (b, h, 0, 0)),
            pl.BlockSpec((DH, NH * 16), lambda b, h: (0, 0)),
        ],
        out_specs=pl.BlockSpec((1, 1, NH, S, 1), lambda b, h: (b, h, 0, 0, 0)),
        out_shape=jax.ShapeDtypeStruct((B, H, NH, S, 1), jnp.int32),
    )(qkvp, rot2)


def _attention(sorted_rows):
    def body(cur_ref, prev_ref, o_ref):
        q = cur_ref[0, :, 0:64]
        kk = jnp.concatenate([prev_ref[0, :, 64:128], cur_ref[0, :, 64:128]], 0)
        vv = jnp.concatenate([prev_ref[0, :, 128:192], cur_ref[0, :, 128:192]], 0)
        mq = cur_ref[0, :, 192:256]
        mk = jnp.concatenate([prev_ref[0, :, 192:256], cur_ref[0, :, 192:256]], 0)
        lane = lax.broadcasted_iota(jnp.int32, (1, 64), 1)
        sel_pad = (lane == 0).astype(jnp.float32)
        sel_pos = (lane == 1).astype(jnp.float32)
        dg = lambda a, bb, prec: lax.dot_general(
            a, bb, (((1,), (1,)), ((), ())), precision=prec)
        pad_row = dg(sel_pad, mk, HI)
        pos_row = dg(sel_pos, mk, HI)
        qpos = mq[:, 1:2]
        dots = dg(q, kk, None) * (1.0 / math.sqrt(DH))
        dots = jnp.where(qpos == pos_row, dots - 1e5, dots)
        dots = jnp.where(pad_row > 0.5, -1e9, dots)
        mx = jnp.max(dots, axis=1, keepdims=True)
        ex = jnp.exp(dots - mx)
        sm = jnp.sum(ex, axis=1, keepdims=True)
        lse = mx + jnp.log(sm)
        bo = jnp.dot(ex, vv) / sm
        lane2 = lax.broadcasted_iota(jnp.int32, (BK, 64), 1)
        extra = jnp.where(lane2 == 0, lse, 0.0)
        o_ref[0] = jnp.concatenate([bo, extra], axis=1)

    return pl.pallas_call(
        body,
        grid=(NR, NC_CHUNKS),
        in_specs=[
            pl.BlockSpec((1, BK, RW), lambda n, c: (n, c, 0)),
            pl.BlockSpec((1, BK, RW), lambda n, c: (n, (c + NC_CHUNKS - 1) % NC_CHUNKS, 0)),
        ],
        out_specs=pl.BlockSpec((1, BK, OW), lambda n, c: (n, c, 0)),
        out_shape=jax.ShapeDtypeStruct((NR, S, OW), jnp.float32),
    )(sorted_rows.reshape(NR, S, RW), sorted_rows.reshape(NR, S, RW))


def _combine(uo, x, p):
    def body(uo_ref, x_ref, wo_ref, o_ref):
        parts = []
        for h in range(H):
            ls = jnp.concatenate(
                [uo_ref[0, h, j, :, 64:65] for j in range(NH)], axis=1)
            mx = jnp.max(ls, axis=1, keepdims=True)
            e = jnp.exp(ls - mx)
            w = e / jnp.sum(e, axis=1, keepdims=True)
            acc = w[:, 0:1] * uo_ref[0, h, 0, :, 0:64]
            for j in range(1, NH):
                acc = acc + w[:, j:j + 1] * uo_ref[0, h, j, :, 0:64]
            parts.append(acc)
        attn = jnp.concatenate(parts, axis=1)
        o_ref[0] = x_ref[0] + jnp.dot(attn, wo_ref[...])

    return pl.pallas_call(
        body,
        grid=(B, S // TS),
        in_specs=[
            pl.BlockSpec((1, H, NH, TS, OW), lambda b, s: (b, 0, 0, s, 0)),
            pl.BlockSpec((1, TS, D), lambda b, s: (b, s, 0)),
            pl.BlockSpec((D, D), lambda b, s: (0, 0)),
        ],
        out_specs=pl.BlockSpec((1, TS, D), lambda b, s: (b, s, 0)),
        out_shape=jax.ShapeDtypeStruct((B, S, D), jnp.float32),
    )(uo, x, p['Wo'])


def _ffn(x, p):
    FC = 512
    KF = (4 * D) // FC

    def body(x_ref, g_ref, b_ref, w1_ref, b1_ref, w2_ref, b2_ref, o_ref):
        k = pl.program_id(2)
        xb = x_ref[0]
        hn = _layer_norm(xb, g_ref[...], b_ref[...])
        a = jax.nn.gelu(jnp.dot(hn, w1_ref[...]) + b1_ref[...])
        part = jnp.dot(a, w2_ref[...])

        @pl.when(k == 0)
        def _():
            o_ref[0] = xb + b2_ref[...] + part

        @pl.when(k != 0)
        def _():
            o_ref[0] = o_ref[0] + part

    return pl.pallas_call(
        body,
        grid=(B, S // TS, KF),
        in_specs=[
            pl.BlockSpec((1, TS, D), lambda b, s, k: (b, s, 0)),
            pl.BlockSpec((1, D), lambda b, s, k: (0, 0)),
            pl.BlockSpec((1, D), lambda b, s, k: (0, 0)),
            pl.BlockSpec((D, FC), lambda b, s, k: (0, k)),
            pl.BlockSpec((1, FC), lambda b, s, k: (0, k)),
            pl.BlockSpec((FC, D), lambda b, s, k: (k, 0)),
            pl.BlockSpec((1, D), lambda b, s, k: (0, 0)),
        ],
        out_specs=pl.BlockSpec((1, TS, D), lambda b, s, k: (b, s, 0)),
        out_shape=jax.ShapeDtypeStruct((B, S, D), jnp.float32),
    )(x, p['ln2_g'].reshape(1, D), p['ln2_b'].reshape(1, D),
      p['W1'], p['b1f'].reshape(1, 4 * D), p['W2'], p['b2f'].reshape(1, D))


def _head(x, srcf, params):
    def body(x_ref, srcf_ref, wp_ref, bp_ref, wc_ref, bc_ref, o_ref):
        rows = []
        for b in range(B):
            kb = (srcf_ref[b] != 0.0).astype(jnp.float32)
            pooled = lax.dot_general(kb, x_ref[b],
                                     (((0,), (0,)), ((), ())),
                                     precision=HI)
            rows.append(pooled / jnp.sum(kb, axis=0, keepdims=True))
        pool = jnp.concatenate(rows, axis=0)
        hp = jax.nn.relu(jnp.dot(pool, wp_ref[...], precision=HI) + bp_ref[...])
        logits = jnp.dot(hp, wc_ref[...], precision=HI) + bc_ref[...]
        padded = jnp.concatenate([logits, jnp.zeros((B, 64 - NCLS), jnp.float32)], 1)
        o_ref[...] = jnp.concatenate([padded, jnp.zeros((8 - B, 64), jnp.float32)], 0)

    out = pl.pallas_call(
        body,
        in_specs=[
            pl.BlockSpec((B, S, D), lambda: (0, 0, 0)),
            pl.BlockSpec((B, S, 1), lambda: (0, 0, 0)),
            pl.BlockSpec((D, D), lambda: (0, 0)),
            pl.BlockSpec((1, D), lambda: (0, 0)),
            pl.BlockSpec((D, NCLS), lambda: (0, 0)),
            pl.BlockSpec((1, NCLS), lambda: (0, 0)),
        ],
        out_specs=pl.BlockSpec((8, 64), lambda: (0, 0)),
        out_shape=jax.ShapeDtypeStruct((8, 64), jnp.float32),
        grid=(),
    )(x, srcf, params['Wp'], params['bp'].reshape(1, D),
      params['Wc'], params['bc'].reshape(1, NCLS))
    return out[:B, :NCLS]


def kernel(src, source_lengths, params):
    src = src.astype(jnp.int32)
    srcf = src.astype(jnp.float32).reshape(B, S, 1)
    emb_rows = _emb_gather(params['emb'], src.reshape(B * S))
    x = emb_rows.reshape(B, S, D) + PE[None]
    for p in params['layers']:
        qkvp = _qkv_rows(x, srcf, p)
        undo = _ranks(qkvp, p['rotations'].reshape(DH, NH * 16))
        undo_flat = undo.reshape(SROWS)
        srt = _sc_scatter(qkvp.reshape(ROWS, RW), undo_flat)
        so = _attention(srt)
        uo = _sc_gather(so.reshape(SROWS, OW), undo_flat)
        x = _combine(uo.reshape(B, H, NH, S, OW), x, p)
        x = _ffn(x, p)
    return _head(x, srcf, params)
